# Initial kernel scaffold; baseline (speedup 1.0000x reference)
import functools

import jax
import jax.numpy as jnp
from jax import lax
from jax.experimental import pallas as pl
from jax.experimental.pallas import tpu as pltpu
from jax.experimental.pallas import tpu_sc as plsc

N = 10000
E = 320000
D = 128
NSUB = 16
EPT = E // NSUB
CH = 80
NCHUNK = EPT // CH
RPT = N // NSUB
DEGW = 8

_mesh = plsc.VectorSubcoreMesh(core_axis_name="c", subcore_axis_name="s")


@functools.partial(
    pl.kernel,
    out_type=(
        jax.ShapeDtypeStruct((N, DEGW), jnp.float32),
        jax.ShapeDtypeStruct((N, DEGW), jnp.float32),
    ),
    mesh=_mesh,
    scratch_types=(
        pltpu.VMEM((CH,), jnp.int32),
        pltpu.VMEM((CH, DEGW), jnp.float32),
        pltpu.VMEM_SHARED((N, DEGW), jnp.float32),
        pltpu.SemaphoreType.DMA,
    ),
)
def _deg_kernel(dst_s, dst_d, ones_hbm, zeros_hbm, deg_s, deg_d,
                idx, ones, acc, sem):
    c = lax.axis_index("c")
    s = lax.axis_index("s")
    pltpu.sync_copy(ones_hbm, ones)
    pltpu.sync_copy(zeros_hbm.at[pl.ds(s * RPT, RPT)],
                    acc.at[pl.ds(s * RPT, RPT)])
    plsc.subcore_barrier()

    def run(dst_hbm):
        base = s * EPT

        @pl.loop(0, NCHUNK)
        def _chunk(ci):
            off = base + ci * CH
            pltpu.sync_copy(dst_hbm.at[pl.ds(off, CH)], idx)
            pltpu.sync_copy(ones, acc.at[idx], add=True)

    @pl.when(c == 0)
    def _():
        run(dst_s)

    @pl.when(c == 1)
    def _():
        run(dst_d)

    plsc.subcore_barrier()

    @pl.when(c == 0)
    def _():
        pltpu.sync_copy(acc.at[pl.ds(s * RPT, RPT)],
                        deg_s.at[pl.ds(s * RPT, RPT)])

    @pl.when(c == 1)
    def _():
        pltpu.sync_copy(acc.at[pl.ds(s * RPT, RPT)],
                        deg_d.at[pl.ds(s * RPT, RPT)])


@functools.partial(
    pl.kernel,
    out_type=(
        jax.ShapeDtypeStruct((N, D), jnp.float32),
        jax.ShapeDtypeStruct((N, D), jnp.float32),
    ),
    mesh=_mesh,
    scratch_types=(
        pltpu.VMEM((CH,), jnp.int32),
        pltpu.VMEM((CH,), jnp.int32),
        pltpu.VMEM((CH, D), jnp.float32),
        pltpu.VMEM_SHARED((N, D), jnp.float32),
        pltpu.SemaphoreType.DMA,
    ),
)
def _scatter_kernel(src_s, dst_s, src_d, dst_d, gs, gd, zeros_hbm,
                    out_s, out_d, idxs, idxd, rows, acc, sem):
    c = lax.axis_index("c")
    s = lax.axis_index("s")
    pltpu.sync_copy(zeros_hbm.at[pl.ds(s * RPT, RPT)],
                    acc.at[pl.ds(s * RPT, RPT)])
    plsc.subcore_barrier()

    def run(src_hbm, dst_hbm, g_hbm):
        base = s * EPT

        @pl.loop(0, NCHUNK)
        def _chunk(ci):
            off = base + ci * CH
            pltpu.sync_copy(src_hbm.at[pl.ds(off, CH)], idxs)
            pltpu.sync_copy(dst_hbm.at[pl.ds(off, CH)], idxd)
            pltpu.async_copy(g_hbm.at[idxs], rows, sem).wait()
            pltpu.sync_copy(rows, acc.at[idxd], add=True)

    @pl.when(c == 0)
    def _():
        run(src_s, dst_s, gs)

    @pl.when(c == 1)
    def _():
        run(src_d, dst_d, gd)

    plsc.subcore_barrier()

    @pl.when(c == 0)
    def _():
        pltpu.sync_copy(acc.at[pl.ds(s * RPT, RPT)],
                        out_s.at[pl.ds(s * RPT, RPT)])

    @pl.when(c == 1)
    def _():
        pltpu.sync_copy(acc.at[pl.ds(s * RPT, RPT)],
                        out_d.at[pl.ds(s * RPT, RPT)])


BR = 2000
_f32 = jnp.float32


def _dis(deg_block):
    return lax.rsqrt(deg_block[:, 0:1] + 1.0)


def _tc_pre_body(x_ref, ws_ref, wd_ref, degs_ref, degd_ref, gs_ref, gd_ref):
    xb = x_ref[...]
    gs_ref[...] = jnp.dot(xb, ws_ref[...], preferred_element_type=_f32) * _dis(degs_ref[...])
    gd_ref[...] = jnp.dot(xb, wd_ref[...], preferred_element_type=_f32) * _dis(degd_ref[...])


def _tc_mid_body(ss_ref, sd_ref, gs_ref, gd_ref, degs_ref, degd_ref,
                 bs_ref, bd_ref, ws2_ref, wd2_ref, gs2_ref, gd2_ref):
    dis_s = _dis(degs_ref[...])
    dis_d = _dis(degd_ref[...])
    h = (ss_ref[...] + gs_ref[...]) * dis_s + bs_ref[...] \
        + (sd_ref[...] + gd_ref[...]) * dis_d + bd_ref[...]
    h = jnp.maximum(h, 0.0)
    gs2_ref[...] = jnp.dot(h, ws2_ref[...], preferred_element_type=_f32) * dis_s
    gd2_ref[...] = jnp.dot(h, wd2_ref[...], preferred_element_type=_f32) * dis_d


def _tc_fin_body(ss_ref, sd_ref, gs_ref, gd_ref, degs_ref, degd_ref,
                 bs_ref, bd_ref, wl_ref, bl_ref, out_ref):
    dis_s = _dis(degs_ref[...])
    dis_d = _dis(degd_ref[...])
    h = (ss_ref[...] + gs_ref[...]) * dis_s + bs_ref[...] \
        + (sd_ref[...] + gd_ref[...]) * dis_d + bd_ref[...]
    h = jnp.maximum(h, 0.0)
    out_ref[...] = jnp.dot(h, wl_ref[...], preferred_element_type=_f32) + bl_ref[...]


def _row_spec(w):
    return pl.BlockSpec((BR, w), lambda i: (i, 0))


def _full_spec(shape):
    return pl.BlockSpec(shape, lambda i: (0, 0))


_mat = _row_spec(D)
_deg = _row_spec(DEGW)
_w = _full_spec((D, D))
_b = _full_spec((1, D))

_tc_pre = pl.pallas_call(
    _tc_pre_body,
    grid=(N // BR,),
    in_specs=[_mat, _w, _w, _deg, _deg],
    out_specs=[_mat, _mat],
    out_shape=[jax.ShapeDtypeStruct((N, D), _f32)] * 2,
)

_tc_mid = pl.pallas_call(
    _tc_mid_body,
    grid=(N // BR,),
    in_specs=[_mat, _mat, _mat, _mat, _deg, _deg, _b, _b, _w, _w],
    out_specs=[_mat, _mat],
    out_shape=[jax.ShapeDtypeStruct((N, D), _f32)] * 2,
)

_tc_fin = pl.pallas_call(
    _tc_fin_body,
    grid=(N // BR,),
    in_specs=[_mat, _mat, _mat, _mat, _deg, _deg, _b, _b, _w, _b],
    out_specs=_mat,
    out_shape=jax.ShapeDtypeStruct((N, D), _f32),
)


def kernel(x, edge_index_spring, edge_index_damper,
           Ws1, bs1, Wd1, bd1, Ws2, bs2, Wd2, bd2, Wl, bl):
    src_s, dst_s = edge_index_spring[0], edge_index_spring[1]
    src_d, dst_d = edge_index_damper[0], edge_index_damper[1]

    ones8 = jnp.ones((CH, DEGW), _f32)
    zeros8 = jnp.zeros((N, DEGW), _f32)
    zerosD = jnp.zeros((N, D), _f32)

    deg_s, deg_d = _deg_kernel(dst_s, dst_d, ones8, zeros8)

    gs1, gd1 = _tc_pre(x, Ws1, Wd1, deg_s, deg_d)
    ss1, sd1 = _scatter_kernel(src_s, dst_s, src_d, dst_d, gs1, gd1, zerosD)

    gs2, gd2 = _tc_mid(ss1, sd1, gs1, gd1, deg_s, deg_d,
                       bs1.reshape(1, D), bd1.reshape(1, D), Ws2, Wd2)
    ss2, sd2 = _scatter_kernel(src_s, dst_s, src_d, dst_d, gs2, gd2, zerosD)

    out = _tc_fin(ss2, sd2, gs2, gd2, deg_s, deg_d,
                  bs2.reshape(1, D), bd2.reshape(1, D), Wl, bl.reshape(1, D))
    return out


# speedup vs baseline: 12.3549x; 12.3549x over previous
import functools

import jax
import jax.numpy as jnp
from jax import lax
from jax.experimental import pallas as pl
from jax.experimental.pallas import tpu as pltpu
from jax.experimental.pallas import tpu_sc as plsc

N = 10000
E = 320000
D = 128
NSUB = 16
EPT = E // NSUB
CH = 80
NCHUNK = EPT // CH
ROWS_A = 624
ROWS_B = N - (NSUB - 1) * ROWS_A

_mesh = plsc.VectorSubcoreMesh(core_axis_name="c", subcore_axis_name="s")


def _tile_rows_copy(src, dst, s):
    @pl.when(s < NSUB - 1)
    def _():
        sl = pl.ds(s * ROWS_A, ROWS_A)
        pltpu.sync_copy(src.at[sl], dst.at[sl])

    @pl.when(s == NSUB - 1)
    def _():
        sl = pl.ds((NSUB - 1) * ROWS_A, ROWS_B)
        pltpu.sync_copy(src.at[sl], dst.at[sl])


@functools.partial(
    pl.kernel,
    out_type=(
        jax.ShapeDtypeStruct((N,), jnp.float32),
        jax.ShapeDtypeStruct((N,), jnp.float32),
    ),
    mesh=_mesh,
    scratch_types=(
        pltpu.VMEM((CH,), jnp.int32),
        pltpu.VMEM((CH,), jnp.float32),
        pltpu.VMEM((ROWS_B,), jnp.float32),
        pltpu.VMEM_SHARED((N,), jnp.float32),
        pltpu.SemaphoreType.DMA,
    ),
)
def _deg_kernel(dst_s, dst_d, deg_s, deg_d, idx, ones, stage, acc, sem):
    c = lax.axis_index("c")
    s = lax.axis_index("s")

    @pl.loop(0, CH // 16)
    def _fill_ones(i):
        ones[pl.ds(i * 16, 16)] = jnp.ones((16,), jnp.float32)

    @pl.loop(0, ROWS_B // 16)
    def _fill_zero(i):
        stage[pl.ds(i * 16, 16)] = jnp.zeros((16,), jnp.float32)

    @pl.when(s < NSUB - 1)
    def _():
        pltpu.sync_copy(stage.at[pl.ds(0, ROWS_A)],
                        acc.at[pl.ds(s * ROWS_A, ROWS_A)])

    @pl.when(s == NSUB - 1)
    def _():
        pltpu.sync_copy(stage, acc.at[pl.ds((NSUB - 1) * ROWS_A, ROWS_B)])

    plsc.subcore_barrier()

    def run(dst_hbm):
        base = s * EPT

        @pl.loop(0, NCHUNK)
        def _chunk(ci):
            off = base + ci * CH
            pltpu.sync_copy(dst_hbm.at[pl.ds(off, CH)], idx)
            pltpu.sync_copy(ones, acc.at[idx], add=True)

    @pl.when(c == 0)
    def _():
        run(dst_s)

    @pl.when(c == 1)
    def _():
        run(dst_d)

    plsc.subcore_barrier()

    def copy_out(out_hbm):
        @pl.when(s < NSUB - 1)
        def _():
            sl = pl.ds(s * ROWS_A, ROWS_A)
            pltpu.sync_copy(acc.at[sl], stage.at[pl.ds(0, ROWS_A)])
            pltpu.sync_copy(stage.at[pl.ds(0, ROWS_A)], out_hbm.at[sl])

        @pl.when(s == NSUB - 1)
        def _():
            sl = pl.ds((NSUB - 1) * ROWS_A, ROWS_B)
            pltpu.sync_copy(acc.at[sl], stage)
            pltpu.sync_copy(stage, out_hbm.at[sl])

    @pl.when(c == 0)
    def _():
        copy_out(deg_s)

    @pl.when(c == 1)
    def _():
        copy_out(deg_d)


@functools.partial(
    pl.kernel,
    out_type=(
        jax.ShapeDtypeStruct((N, D), jnp.float32),
        jax.ShapeDtypeStruct((N, D), jnp.float32),
    ),
    mesh=_mesh,
    scratch_types=(
        pltpu.VMEM((CH,), jnp.int32),
        pltpu.VMEM((CH,), jnp.int32),
        pltpu.VMEM((CH, D), jnp.float32),
        pltpu.VMEM_SHARED((N, D), jnp.float32),
        pltpu.SemaphoreType.DMA,
    ),
)
def _scatter_kernel(src_s, dst_s, src_d, dst_d, gs, gd, zeros_hbm,
                    out_s, out_d, idxs, idxd, rows, acc, sem):
    c = lax.axis_index("c")
    s = lax.axis_index("s")
    _tile_rows_copy(zeros_hbm, acc, s)
    plsc.subcore_barrier()

    def run(src_hbm, dst_hbm, g_hbm):
        base = s * EPT

        @pl.loop(0, NCHUNK)
        def _chunk(ci):
            off = base + ci * CH
            pltpu.sync_copy(src_hbm.at[pl.ds(off, CH)], idxs)
            pltpu.sync_copy(dst_hbm.at[pl.ds(off, CH)], idxd)
            pltpu.async_copy(g_hbm.at[idxs], rows, sem).wait()
            pltpu.sync_copy(rows, acc.at[idxd], add=True)

    @pl.when(c == 0)
    def _():
        run(src_s, dst_s, gs)

    @pl.when(c == 1)
    def _():
        run(src_d, dst_d, gd)

    plsc.subcore_barrier()

    @pl.when(c == 0)
    def _():
        _tile_rows_copy(acc, out_s, s)

    @pl.when(c == 1)
    def _():
        _tile_rows_copy(acc, out_d, s)


BR = 2000
_f32 = jnp.float32


def _dis(deg_block):
    return lax.rsqrt(deg_block + 1.0)


def _tc_pre_body(x_ref, ws_ref, wd_ref, degs_ref, degd_ref, gs_ref, gd_ref):
    xb = x_ref[...]
    gs_ref[...] = jnp.dot(xb, ws_ref[...], preferred_element_type=_f32) * _dis(degs_ref[...])
    gd_ref[...] = jnp.dot(xb, wd_ref[...], preferred_element_type=_f32) * _dis(degd_ref[...])


def _tc_mid_body(ss_ref, sd_ref, gs_ref, gd_ref, degs_ref, degd_ref,
                 bs_ref, bd_ref, ws2_ref, wd2_ref, gs2_ref, gd2_ref):
    dis_s = _dis(degs_ref[...])
    dis_d = _dis(degd_ref[...])
    h = (ss_ref[...] + gs_ref[...]) * dis_s + bs_ref[...] \
        + (sd_ref[...] + gd_ref[...]) * dis_d + bd_ref[...]
    h = jnp.maximum(h, 0.0)
    gs2_ref[...] = jnp.dot(h, ws2_ref[...], preferred_element_type=_f32) * dis_s
    gd2_ref[...] = jnp.dot(h, wd2_ref[...], preferred_element_type=_f32) * dis_d


def _tc_fin_body(ss_ref, sd_ref, gs_ref, gd_ref, degs_ref, degd_ref,
                 bs_ref, bd_ref, wl_ref, bl_ref, out_ref):
    dis_s = _dis(degs_ref[...])
    dis_d = _dis(degd_ref[...])
    h = (ss_ref[...] + gs_ref[...]) * dis_s + bs_ref[...] \
        + (sd_ref[...] + gd_ref[...]) * dis_d + bd_ref[...]
    h = jnp.maximum(h, 0.0)
    out_ref[...] = jnp.dot(h, wl_ref[...], preferred_element_type=_f32) + bl_ref[...]


def _row_spec(w):
    return pl.BlockSpec((BR, w), lambda i: (i, 0))


def _full_spec(shape):
    return pl.BlockSpec(shape, lambda i: (0, 0))


_mat = _row_spec(D)
_deg = _row_spec(1)
_w = _full_spec((D, D))
_b = _full_spec((1, D))

_tc_pre = pl.pallas_call(
    _tc_pre_body,
    grid=(N // BR,),
    in_specs=[_mat, _w, _w, _deg, _deg],
    out_specs=[_mat, _mat],
    out_shape=[jax.ShapeDtypeStruct((N, D), _f32)] * 2,
)

_tc_mid = pl.pallas_call(
    _tc_mid_body,
    grid=(N // BR,),
    in_specs=[_mat, _mat, _mat, _mat, _deg, _deg, _b, _b, _w, _w],
    out_specs=[_mat, _mat],
    out_shape=[jax.ShapeDtypeStruct((N, D), _f32)] * 2,
)

_tc_fin = pl.pallas_call(
    _tc_fin_body,
    grid=(N // BR,),
    in_specs=[_mat, _mat, _mat, _mat, _deg, _deg, _b, _b, _w, _b],
    out_specs=_mat,
    out_shape=jax.ShapeDtypeStruct((N, D), _f32),
)


def kernel(x, edge_index_spring, edge_index_damper,
           Ws1, bs1, Wd1, bd1, Ws2, bs2, Wd2, bd2, Wl, bl):
    src_s, dst_s = edge_index_spring[0], edge_index_spring[1]
    src_d, dst_d = edge_index_damper[0], edge_index_damper[1]

    zerosD = jnp.zeros((N, D), _f32)

    deg_s, deg_d = _deg_kernel(dst_s, dst_d)
    deg_s = deg_s.reshape(N, 1)
    deg_d = deg_d.reshape(N, 1)

    gs1, gd1 = _tc_pre(x, Ws1, Wd1, deg_s, deg_d)
    ss1, sd1 = _scatter_kernel(src_s, dst_s, src_d, dst_d, gs1, gd1, zerosD)

    gs2, gd2 = _tc_mid(ss1, sd1, gs1, gd1, deg_s, deg_d,
                       bs1.reshape(1, D), bd1.reshape(1, D), Ws2, Wd2)
    ss2, sd2 = _scatter_kernel(src_s, dst_s, src_d, dst_d, gs2, gd2, zerosD)

    out = _tc_fin(ss2, sd2, gs2, gd2, deg_s, deg_d,
                  bs2.reshape(1, D), bd2.reshape(1, D), Wl, bl.reshape(1, D))
    return out


# speedup vs baseline: 21.4237x; 1.7340x over previous
import functools

import jax
import jax.numpy as jnp
from jax import lax
from jax.experimental import pallas as pl
from jax.experimental.pallas import tpu as pltpu
from jax.experimental.pallas import tpu_sc as plsc

N = 10000
E = 320000
D = 128
NSUB = 16
EPT = E // NSUB
CH = 80
NCHUNK = EPT // CH
SEPT = EPT // 2
SBCH = SEPT // CH
ROWS_A = 624
ROWS_B = N - (NSUB - 1) * ROWS_A

_mesh = plsc.VectorSubcoreMesh(core_axis_name="c", subcore_axis_name="s")


def _tile_rows_copy(src, dst, s):
    @pl.when(s < NSUB - 1)
    def _():
        sl = pl.ds(s * ROWS_A, ROWS_A)
        pltpu.sync_copy(src.at[sl], dst.at[sl])

    @pl.when(s == NSUB - 1)
    def _():
        sl = pl.ds((NSUB - 1) * ROWS_A, ROWS_B)
        pltpu.sync_copy(src.at[sl], dst.at[sl])


@functools.partial(
    pl.kernel,
    out_type=(
        jax.ShapeDtypeStruct((N,), jnp.float32),
        jax.ShapeDtypeStruct((N,), jnp.float32),
    ),
    mesh=_mesh,
    scratch_types=(
        pltpu.VMEM((CH,), jnp.int32),
        pltpu.VMEM((CH,), jnp.float32),
        pltpu.VMEM((ROWS_B,), jnp.float32),
        pltpu.VMEM_SHARED((N,), jnp.float32),
        pltpu.SemaphoreType.DMA,
    ),
)
def _deg_kernel(dst_s, dst_d, deg_s, deg_d, idx, ones, stage, acc, sem):
    c = lax.axis_index("c")
    s = lax.axis_index("s")

    @pl.loop(0, CH // 16)
    def _fill_ones(i):
        ones[pl.ds(i * 16, 16)] = jnp.ones((16,), jnp.float32)

    @pl.loop(0, ROWS_B // 16)
    def _fill_zero(i):
        stage[pl.ds(i * 16, 16)] = jnp.zeros((16,), jnp.float32)

    @pl.when(s < NSUB - 1)
    def _():
        pltpu.sync_copy(stage.at[pl.ds(0, ROWS_A)],
                        acc.at[pl.ds(s * ROWS_A, ROWS_A)])

    @pl.when(s == NSUB - 1)
    def _():
        pltpu.sync_copy(stage, acc.at[pl.ds((NSUB - 1) * ROWS_A, ROWS_B)])

    plsc.subcore_barrier()

    def run(dst_hbm):
        base = s * EPT

        @pl.loop(0, NCHUNK)
        def _chunk(ci):
            off = base + ci * CH
            pltpu.sync_copy(dst_hbm.at[pl.ds(off, CH)], idx)
            pltpu.sync_copy(ones, acc.at[idx], add=True)

    @pl.when(c == 0)
    def _():
        run(dst_s)

    @pl.when(c == 1)
    def _():
        run(dst_d)

    plsc.subcore_barrier()

    def copy_out(out_hbm):
        @pl.when(s < NSUB - 1)
        def _():
            sl = pl.ds(s * ROWS_A, ROWS_A)
            pltpu.sync_copy(acc.at[sl], stage.at[pl.ds(0, ROWS_A)])
            pltpu.sync_copy(stage.at[pl.ds(0, ROWS_A)], out_hbm.at[sl])

        @pl.when(s == NSUB - 1)
        def _():
            sl = pl.ds((NSUB - 1) * ROWS_A, ROWS_B)
            pltpu.sync_copy(acc.at[sl], stage)
            pltpu.sync_copy(stage, out_hbm.at[sl])

    @pl.when(c == 0)
    def _():
        copy_out(deg_s)

    @pl.when(c == 1)
    def _():
        copy_out(deg_d)


@functools.partial(
    pl.kernel,
    out_type=(
        jax.ShapeDtypeStruct((N, D), jnp.float32),
        jax.ShapeDtypeStruct((N, D), jnp.float32),
    ),
    mesh=_mesh,
    scratch_types=(
        pltpu.VMEM((EPT // 2,), jnp.int32),
        pltpu.VMEM((EPT // 2,), jnp.int32),
        pltpu.VMEM((CH,), jnp.int32),
        pltpu.VMEM((CH,), jnp.int32),
        pltpu.VMEM((CH, D), jnp.float32),
        pltpu.VMEM((CH, D), jnp.float32),
        pltpu.VMEM_SHARED((N, D), jnp.float32),
        pltpu.SemaphoreType.DMA,
    ),
)
def _scatter_kernel(src_s, dst_s, src_d, dst_d, gs, gd, zeros_hbm,
                    out_s, out_d, srcbuf, dstbuf, idxs, idxd,
                    rows0, rows1, acc, sem):
    c = lax.axis_index("c")
    s = lax.axis_index("s")
    _tile_rows_copy(zeros_hbm, acc, s)
    plsc.subcore_barrier()

    def copy_idx(big, ci, small):
        for j in range(CH // 16):
            small[pl.ds(j * 16, 16)] = big[pl.ds(ci * CH + j * 16, 16)]

    def run(src_hbm, dst_hbm, g_hbm):
        def halfstep(ci, rows_wait, rows_fire, g_hbm):
            pltpu.make_async_copy(g_hbm.at[pl.ds(0, CH)], rows_wait, sem).wait()

            @pl.when(ci + 1 < SBCH)
            def _():
                copy_idx(srcbuf, ci + 1, idxs)
                pltpu.async_copy(g_hbm.at[idxs], rows_fire, sem)

            copy_idx(dstbuf, ci, idxd)
            pltpu.sync_copy(rows_wait, acc.at[idxd], add=True)

        @pl.loop(0, 2)
        def _sb(b):
            base = s * EPT + b * SEPT
            pltpu.sync_copy(src_hbm.at[pl.ds(base, SEPT)], srcbuf)
            pltpu.sync_copy(dst_hbm.at[pl.ds(base, SEPT)], dstbuf)

            copy_idx(srcbuf, 0, idxs)
            pltpu.async_copy(g_hbm.at[idxs], rows0, sem)

            @pl.loop(0, SBCH - 1, step=2)
            def _pair(ci0):
                halfstep(ci0, rows0, rows1, g_hbm)
                halfstep(ci0 + 1, rows1, rows0, g_hbm)

            halfstep(SBCH - 1, rows0, rows1, g_hbm)

    @pl.when(c == 0)
    def _():
        run(src_s, dst_s, gs)

    @pl.when(c == 1)
    def _():
        run(src_d, dst_d, gd)

    plsc.subcore_barrier()

    @pl.when(c == 0)
    def _():
        _tile_rows_copy(acc, out_s, s)

    @pl.when(c == 1)
    def _():
        _tile_rows_copy(acc, out_d, s)


BR = 2000
_f32 = jnp.float32


def _dis(deg_block):
    return lax.rsqrt(deg_block + 1.0)


def _tc_pre_body(x_ref, ws_ref, wd_ref, degs_ref, degd_ref, gs_ref, gd_ref):
    xb = x_ref[...]
    gs_ref[...] = jnp.dot(xb, ws_ref[...], preferred_element_type=_f32) * _dis(degs_ref[...])
    gd_ref[...] = jnp.dot(xb, wd_ref[...], preferred_element_type=_f32) * _dis(degd_ref[...])


def _tc_mid_body(ss_ref, sd_ref, gs_ref, gd_ref, degs_ref, degd_ref,
                 bs_ref, bd_ref, ws2_ref, wd2_ref, gs2_ref, gd2_ref):
    dis_s = _dis(degs_ref[...])
    dis_d = _dis(degd_ref[...])
    h = (ss_ref[...] + gs_ref[...]) * dis_s + bs_ref[...] \
        + (sd_ref[...] + gd_ref[...]) * dis_d + bd_ref[...]
    h = jnp.maximum(h, 0.0)
    gs2_ref[...] = jnp.dot(h, ws2_ref[...], preferred_element_type=_f32) * dis_s
    gd2_ref[...] = jnp.dot(h, wd2_ref[...], preferred_element_type=_f32) * dis_d


def _tc_fin_body(ss_ref, sd_ref, gs_ref, gd_ref, degs_ref, degd_ref,
                 bs_ref, bd_ref, wl_ref, bl_ref, out_ref):
    dis_s = _dis(degs_ref[...])
    dis_d = _dis(degd_ref[...])
    h = (ss_ref[...] + gs_ref[...]) * dis_s + bs_ref[...] \
        + (sd_ref[...] + gd_ref[...]) * dis_d + bd_ref[...]
    h = jnp.maximum(h, 0.0)
    out_ref[...] = jnp.dot(h, wl_ref[...], preferred_element_type=_f32) + bl_ref[...]


def _row_spec(w):
    return pl.BlockSpec((BR, w), lambda i: (i, 0))


def _full_spec(shape):
    return pl.BlockSpec(shape, lambda i: (0, 0))


_mat = _row_spec(D)
_deg = _row_spec(1)
_w = _full_spec((D, D))
_b = _full_spec((1, D))

_tc_pre = pl.pallas_call(
    _tc_pre_body,
    grid=(N // BR,),
    in_specs=[_mat, _w, _w, _deg, _deg],
    out_specs=[_mat, _mat],
    out_shape=[jax.ShapeDtypeStruct((N, D), _f32)] * 2,
)

_tc_mid = pl.pallas_call(
    _tc_mid_body,
    grid=(N // BR,),
    in_specs=[_mat, _mat, _mat, _mat, _deg, _deg, _b, _b, _w, _w],
    out_specs=[_mat, _mat],
    out_shape=[jax.ShapeDtypeStruct((N, D), _f32)] * 2,
)

_tc_fin = pl.pallas_call(
    _tc_fin_body,
    grid=(N // BR,),
    in_specs=[_mat, _mat, _mat, _mat, _deg, _deg, _b, _b, _w, _b],
    out_specs=_mat,
    out_shape=jax.ShapeDtypeStruct((N, D), _f32),
)


def kernel(x, edge_index_spring, edge_index_damper,
           Ws1, bs1, Wd1, bd1, Ws2, bs2, Wd2, bd2, Wl, bl):
    src_s, dst_s = edge_index_spring[0], edge_index_spring[1]
    src_d, dst_d = edge_index_damper[0], edge_index_damper[1]

    zerosD = jnp.zeros((N, D), _f32)

    deg_s, deg_d = _deg_kernel(dst_s, dst_d)
    deg_s = deg_s.reshape(N, 1)
    deg_d = deg_d.reshape(N, 1)

    gs1, gd1 = _tc_pre(x, Ws1, Wd1, deg_s, deg_d)
    ss1, sd1 = _scatter_kernel(src_s, dst_s, src_d, dst_d, gs1, gd1, zerosD)

    gs2, gd2 = _tc_mid(ss1, sd1, gs1, gd1, deg_s, deg_d,
                       bs1.reshape(1, D), bd1.reshape(1, D), Ws2, Wd2)
    ss2, sd2 = _scatter_kernel(src_s, dst_s, src_d, dst_d, gs2, gd2, zerosD)

    out = _tc_fin(ss2, sd2, gs2, gd2, deg_s, deg_d,
                  bs2.reshape(1, D), bd2.reshape(1, D), Wl, bl.reshape(1, D))
    return out
